# baseline (speedup 1.0000x reference)
import functools

import jax
import jax.numpy as jnp
from jax import lax
from jax.experimental import pallas as pl
from jax.experimental.pallas import tpu as pltpu
from jax.experimental.pallas import tpu_sc as plsc


def _make_sc_gather(K, N):
    NC = 1
    rpc = N // NC
    mesh = plsc.ScalarSubcoreMesh(axis_name="c", num_cores=NC)

    @functools.partial(
        pl.kernel,
        mesh=mesh,
        out_type=jax.ShapeDtypeStruct((N, N), jnp.float32),
        scratch_types=[
            pltpu.SMEM((K,), jnp.float32),
            pltpu.VMEM_SHARED((N // NC, N), jnp.float32),
            pltpu.SemaphoreType.DMA,
            pltpu.SemaphoreType.DMA,
        ],
        compiler_params=pltpu.CompilerParams(needs_layout_passes=False),
    )
    def sc_gather(adj_hbm, z_hbm, out_hbm, z_s, sp, sem, sem2):
        cid = lax.axis_index("c")
        pltpu.sync_copy(z_hbm, z_s)
        best_val = z_s[0]
        best_idx = jnp.int32(0)
        for i in range(1, K):
            v = z_s[i]
            gt = v > best_val
            best_idx = jnp.where(gt, jnp.int32(i), best_idx)
            best_val = jnp.maximum(best_val, v)
        n_ch = 4
        rows = rpc // n_ch
        gets = []
        for j in range(n_ch):
            gets.append(
                pltpu.async_copy(
                    adj_hbm.at[best_idx, pl.ds(cid * rpc + j * rows, rows)],
                    sp.at[pl.ds(j * rows, rows)],
                    sem,
                )
            )
        puts = []
        for j in range(n_ch):
            gets[j].wait()
            puts.append(
                pltpu.async_copy(
                    sp.at[pl.ds(j * rows, rows)],
                    out_hbm.at[pl.ds(cid * rpc + j * rows, rows)],
                    sem2,
                )
            )
        for p in puts:
            p.wait()

    return sc_gather


def kernel(adj_matrices):
    K, N, _ = adj_matrices.shape
    z = jnp.ones((K,), jnp.float32) + jax.random.gumbel(
        jax.random.key(42), (K,), jnp.float32
    )
    return _make_sc_gather(K, N)(adj_matrices, z)


# speedup vs baseline: 1.0040x; 1.0040x over previous
import functools

import jax
import jax.numpy as jnp
from jax import lax
from jax.experimental import pallas as pl
from jax.experimental.pallas import tpu as pltpu
from jax.experimental.pallas import tpu_sc as plsc


def _make_sc_gather(K, N):
    NC = 1
    rpc = N // NC
    mesh = plsc.ScalarSubcoreMesh(axis_name="c", num_cores=NC)

    @functools.partial(
        pl.kernel,
        mesh=mesh,
        out_type=jax.ShapeDtypeStruct((N, N), jnp.float32),
        scratch_types=[
            pltpu.SMEM((K,), jnp.float32),
            pltpu.VMEM_SHARED((N // NC, N), jnp.float32),
            pltpu.SemaphoreType.DMA,
            pltpu.SemaphoreType.DMA,
        ],
        compiler_params=pltpu.CompilerParams(),
    )
    def sc_gather(adj_hbm, z_hbm, out_hbm, z_s, sp, sem, sem2):
        cid = lax.axis_index("c")
        pltpu.sync_copy(z_hbm, z_s)
        best_val = z_s[0]
        best_idx = jnp.int32(0)
        for i in range(1, K):
            v = z_s[i]
            gt = v > best_val
            best_idx = jnp.where(gt, jnp.int32(i), best_idx)
            best_val = jnp.maximum(best_val, v)
        n_ch = 4
        rows = rpc // n_ch
        gets = []
        for j in range(n_ch):
            gets.append(
                pltpu.async_copy(
                    adj_hbm.at[best_idx, pl.ds(cid * rpc + j * rows, rows)],
                    sp.at[pl.ds(j * rows, rows)],
                    sem,
                )
            )
        puts = []
        for j in range(n_ch):
            gets[j].wait()
            puts.append(
                pltpu.async_copy(
                    sp.at[pl.ds(j * rows, rows)],
                    out_hbm.at[pl.ds(cid * rpc + j * rows, rows)],
                    sem2,
                )
            )
        for p in puts:
            p.wait()

    return sc_gather


def kernel(adj_matrices):
    K, N, _ = adj_matrices.shape
    z = jnp.ones((K,), jnp.float32) + jax.random.gumbel(
        jax.random.key(42), (K,), jnp.float32
    )
    return _make_sc_gather(K, N)(adj_matrices, z)


# speedup vs baseline: 1.0103x; 1.0063x over previous
import functools

import jax
import jax.numpy as jnp
from jax import lax
from jax.experimental import pallas as pl
from jax.experimental.pallas import tpu as pltpu
from jax.experimental.pallas import tpu_sc as plsc


def _make_sc_gather(K, N):
    NC = 1
    rpc = N // NC
    mesh = plsc.ScalarSubcoreMesh(axis_name="c", num_cores=NC)

    @functools.partial(
        pl.kernel,
        mesh=mesh,
        out_type=jax.ShapeDtypeStruct((N, N), jnp.float32),
        scratch_types=[
            pltpu.SMEM((K // 2,), jnp.float32),
            pltpu.SMEM((K // 2,), jnp.float32),
            pltpu.VMEM_SHARED((N // NC, N), jnp.float32),
            pltpu.SemaphoreType.DMA,
            pltpu.SemaphoreType.DMA,
        ],
        compiler_params=pltpu.CompilerParams(
            needs_layout_passes=False, use_tc_tiling_on_sc=True
        ),
    )
    def sc_gather(adj_hbm, z_hbm, out_hbm, z_a, z_b, sp, sem, sem2):
        cid = lax.axis_index("c")
        H = K // 2
        get_a = pltpu.async_copy(z_hbm.at[pl.ds(0, H)], z_a, sem)
        get_b = pltpu.async_copy(z_hbm.at[pl.ds(H, H)], z_b, sem2)
        get_a.wait()
        best_val = z_a[0]
        best_idx = jnp.int32(0)
        for i in range(1, H):
            v = z_a[i]
            gt = v > best_val
            best_idx = jnp.where(gt, jnp.int32(i), best_idx)
            best_val = jnp.maximum(best_val, v)
        get_b.wait()
        for i in range(H):
            v = z_b[i]
            gt = v > best_val
            best_idx = jnp.where(gt, jnp.int32(H + i), best_idx)
            best_val = jnp.maximum(best_val, v)
        n_ch = 4
        rows = rpc // n_ch
        gets = []
        for j in range(n_ch):
            gets.append(
                pltpu.async_copy(
                    adj_hbm.at[best_idx, pl.ds(cid * rpc + j * rows, rows)],
                    sp.at[pl.ds(j * rows, rows)],
                    sem,
                )
            )
        puts = []
        for j in range(n_ch):
            gets[j].wait()
            puts.append(
                pltpu.async_copy(
                    sp.at[pl.ds(j * rows, rows)],
                    out_hbm.at[pl.ds(cid * rpc + j * rows, rows)],
                    sem2,
                )
            )
        for p in puts:
            p.wait()

    return sc_gather


def kernel(adj_matrices):
    K, N, _ = adj_matrices.shape
    z = jnp.ones((K,), jnp.float32) + jax.random.gumbel(
        jax.random.key(42), (K,), jnp.float32
    )
    return _make_sc_gather(K, N)(adj_matrices, z)
